# Initial kernel scaffold; baseline (speedup 1.0000x reference)
import jax
import jax.numpy as jnp
from jax.experimental import pallas as pl


def kernel(x, edge_index, W_root_0, W_rel_0, b_rel_0, W_root_1, W_rel_1, b_rel_1):
    raise NotImplementedError("write your pallas kernel here")


# speedup vs baseline: 40.4850x; 40.4850x over previous
import functools

import jax
import jax.numpy as jnp
from jax import lax
from jax.experimental import pallas as pl
from jax.experimental.pallas import tpu as pltpu
from jax.experimental.pallas import tpu_sc as plsc

N = 100000
E = 3200000
D = 6
DP = 8
NC = 2
NS = 16
NW = NC * NS
SUB = 128
K = 8
G = 98
EPW = G * K * SUB
EPAD = NW * EPW
NSUB = EPAD // SUB
NX = 100096
XPT = NX // NS
RPT = XPT


def _sc_body(x_hbm, src_hbm, dst_hbm, z_hbm, out_hbm,
             xs, acc, idx_s, idx_d, rows, gsem):
    cid = lax.axis_index("c")
    sid = lax.axis_index("s")
    wid = sid * NC + cid
    pltpu.sync_copy(x_hbm.at[pl.ds(sid * XPT, XPT)],
                    xs.at[pl.ds(sid * XPT, XPT)])
    pltpu.sync_copy(z_hbm, acc.at[pl.ds(sid * RPT, RPT)])
    plsc.subcore_barrier()

    wbase = wid * (G * K)

    def outer(g, carry):
        r0 = wbase + g * K
        pltpu.sync_copy(src_hbm.at[pl.ds(r0, K)], idx_s)
        pltpu.sync_copy(dst_hbm.at[pl.ds(r0, K)], idx_d)
        cps = [pltpu.async_copy(xs.at[idx_s.at[j]], rows.at[j], gsem)
               for j in range(K)]
        for cp in cps:
            cp.wait()
        for j in range(K):
            pltpu.sync_copy(rows.at[j], acc.at[idx_d.at[j]], add=True)
        return carry

    lax.fori_loop(0, G, outer, 0)
    plsc.subcore_barrier()
    pltpu.sync_copy(acc.at[pl.ds(sid * RPT, RPT)],
                    out_hbm.at[cid, pl.ds(sid * RPT, RPT)])


_sc_segsum = functools.partial(
    pl.kernel,
    out_type=jax.ShapeDtypeStruct((NC, NX, DP), jnp.float32),
    mesh=plsc.VectorSubcoreMesh(core_axis_name="c", subcore_axis_name="s",
                                num_cores=NC, num_subcores=NS),
    scratch_types=[
        pltpu.VMEM_SHARED((NX, DP), jnp.float32),
        pltpu.VMEM_SHARED((NX, DP), jnp.float32),
        pltpu.VMEM((K, SUB), jnp.int32),
        pltpu.VMEM((K, SUB), jnp.int32),
        pltpu.VMEM((K, SUB, DP), jnp.float32),
        pltpu.SemaphoreType.DMA,
    ],
    compiler_params=pltpu.CompilerParams(use_tc_tiling_on_sc=False),
)(_sc_body)


_RB = 10000


def _tc_linear_body(p_ref, y_ref, wr_ref, wo_ref, b_ref, o_ref):
    agg = p_ref[0] + p_ref[1]
    o_ref[...] = (
        jnp.dot(agg, wr_ref[...], preferred_element_type=jnp.float32)
        + jnp.dot(y_ref[...], wo_ref[...], preferred_element_type=jnp.float32)
        + b_ref[...]
    )


def _tc_linear(p, y, wr, wo, b):
    return pl.pallas_call(
        _tc_linear_body,
        grid=(N // _RB,),
        in_specs=[
            pl.BlockSpec((NC, _RB, DP), lambda i: (0, i, 0)),
            pl.BlockSpec((_RB, DP), lambda i: (i, 0)),
            pl.BlockSpec((DP, DP), lambda i: (0, 0)),
            pl.BlockSpec((DP, DP), lambda i: (0, 0)),
            pl.BlockSpec((1, DP), lambda i: (0, 0)),
        ],
        out_specs=pl.BlockSpec((_RB, DP), lambda i: (i, 0)),
        out_shape=jax.ShapeDtypeStruct((N, DP), jnp.float32),
    )(p, y, wr, wo, b)


def _tc_head_body(q_ref, h_ref, wr_ref, wo_ref, b_ref, o_ref):
    agg = q_ref[0] + q_ref[1]
    h2 = (jnp.dot(agg, wr_ref[...], preferred_element_type=jnp.float32)
          + jnp.dot(h_ref[...], wo_ref[...], preferred_element_type=jnp.float32)
          + b_ref[...])
    h6 = h2[:, :D]
    m = jnp.max(h6, axis=1, keepdims=True)
    ids = lax.broadcasted_iota(jnp.int32, h6.shape, 1)
    cand = jnp.where(h6 >= m, ids, D)
    sel = jnp.min(cand, axis=1, keepdims=True)
    o_ref[...] = jnp.where(ids == sel, 1.0, 0.0).astype(jnp.float32)


def _tc_head(q, h, wr, wo, b):
    return pl.pallas_call(
        _tc_head_body,
        grid=(N // _RB,),
        in_specs=[
            pl.BlockSpec((NC, _RB, DP), lambda i: (0, i, 0)),
            pl.BlockSpec((_RB, DP), lambda i: (i, 0)),
            pl.BlockSpec((DP, DP), lambda i: (0, 0)),
            pl.BlockSpec((DP, DP), lambda i: (0, 0)),
            pl.BlockSpec((1, DP), lambda i: (0, 0)),
        ],
        out_specs=pl.BlockSpec((_RB, D), lambda i: (i, 0)),
        out_shape=jax.ShapeDtypeStruct((N, D), jnp.float32),
    )(q, h, wr, wo, b)


def _pad_w(w):
    return jnp.pad(w.astype(jnp.float32), ((0, DP - D), (0, DP - D)))


def kernel(x, edge_index, W_root_0, W_rel_0, b_rel_0,
           W_root_1, W_rel_1, b_rel_1):
    src = edge_index[0].astype(jnp.int32)
    dst = edge_index[1].astype(jnp.int32)
    pad_src = jnp.full((EPAD - E,), N, jnp.int32)
    pad_dst = jnp.zeros((EPAD - E,), jnp.int32)
    src2d = jnp.concatenate([src, pad_src]).reshape(NSUB, SUB)
    dst2d = jnp.concatenate([dst, pad_dst]).reshape(NSUB, SUB)

    xe = jnp.pad(x.astype(jnp.float32), ((0, NX - N), (0, DP - D)))
    z = jnp.zeros((RPT, DP), jnp.float32)
    wr0, wo0 = _pad_w(W_rel_0), _pad_w(W_root_0)
    wr1, wo1 = _pad_w(W_rel_1), _pad_w(W_root_1)
    b0 = jnp.pad(b_rel_0.astype(jnp.float32), (0, DP - D)).reshape(1, DP)
    b1 = jnp.pad(b_rel_1.astype(jnp.float32), (0, DP - D)).reshape(1, DP)

    p = _sc_segsum(xe, src2d, dst2d, z)
    h1 = _tc_linear(p, xe[:N], wr0, wo0, b0)
    h1e = jnp.pad(h1, ((0, NX - N), (0, 0)))
    q = _sc_segsum(h1e, src2d, dst2d, z)
    return _tc_head(q, h1, wr1, wo1, b1)


# speedup vs baseline: 43.6631x; 1.0785x over previous
import functools

import jax
import jax.numpy as jnp
from jax import lax
from jax.experimental import pallas as pl
from jax.experimental.pallas import tpu as pltpu
from jax.experimental.pallas import tpu_sc as plsc

N = 100000
E = 3200000
D = 6
DP = 8
NC = 2
NS = 16
NW = NC * NS
SUB = 128
K = 8
NSUB = E // SUB
CPW = NSUB // NW
REM = NSUB % NW
F = CPW // K
XPT = N // NS


def _sc_body(x_hbm, ei_hbm, z_hbm, out_hbm, xs, acc, idx_s, idx_d, rows, gsem):
    cid = lax.axis_index("c")
    sid = lax.axis_index("s")
    wid = sid * NC + cid
    pltpu.sync_copy(x_hbm.at[pl.ds(sid * XPT, XPT)],
                    xs.at[pl.ds(sid * XPT, XPT)])
    pltpu.sync_copy(z_hbm, acc.at[pl.ds(sid * XPT, XPT)])
    plsc.subcore_barrier()

    cnt = CPW + jnp.where(wid < REM, 1, 0)
    off = wid * CPW + jnp.minimum(wid, REM)

    def outer(g, carry):
        r0 = off + g * K
        pltpu.sync_copy(ei_hbm.at[0, pl.ds(r0, K)], idx_s)
        pltpu.sync_copy(ei_hbm.at[1, pl.ds(r0, K)], idx_d)
        cps = [pltpu.async_copy(xs.at[idx_s.at[j]], rows.at[j], gsem)
               for j in range(K)]
        for cp in cps:
            cp.wait()
        for j in range(K):
            pltpu.sync_copy(rows.at[j], acc.at[idx_d.at[j]], add=True)
        return carry

    lax.fori_loop(0, F, outer, 0)

    r0t = off + cnt - K
    th = (F + 1) * K - cnt
    pltpu.sync_copy(ei_hbm.at[0, pl.ds(r0t, K)], idx_s)
    pltpu.sync_copy(ei_hbm.at[1, pl.ds(r0t, K)], idx_d)
    for j in range(K):
        @pl.when(j >= th)
        def _():
            pltpu.async_copy(xs.at[idx_s.at[j]], rows.at[j], gsem).wait()
            pltpu.sync_copy(rows.at[j], acc.at[idx_d.at[j]], add=True)

    plsc.subcore_barrier()
    pltpu.sync_copy(acc.at[pl.ds(sid * XPT, XPT)],
                    out_hbm.at[cid, pl.ds(sid * XPT, XPT)])


_sc_segsum = functools.partial(
    pl.kernel,
    out_type=jax.ShapeDtypeStruct((NC, N, DP), jnp.float32),
    mesh=plsc.VectorSubcoreMesh(core_axis_name="c", subcore_axis_name="s",
                                num_cores=NC, num_subcores=NS),
    scratch_types=[
        pltpu.VMEM_SHARED((N, DP), jnp.float32),
        pltpu.VMEM_SHARED((N, DP), jnp.float32),
        pltpu.VMEM((K, SUB), jnp.int32),
        pltpu.VMEM((K, SUB), jnp.int32),
        pltpu.VMEM((K, SUB, DP), jnp.float32),
        pltpu.SemaphoreType.DMA,
    ],
    compiler_params=pltpu.CompilerParams(use_tc_tiling_on_sc=False),
)(_sc_body)


_RB = 10000


def _tc_linear_body(p_ref, y_ref, wr_ref, wo_ref, b_ref, o_ref):
    agg = p_ref[0] + p_ref[1]
    o_ref[...] = (
        jnp.dot(agg, wr_ref[...], preferred_element_type=jnp.float32)
        + jnp.dot(y_ref[...], wo_ref[...], preferred_element_type=jnp.float32)
        + b_ref[...]
    )


def _tc_linear(p, y, wr, wo, b):
    return pl.pallas_call(
        _tc_linear_body,
        grid=(N // _RB,),
        in_specs=[
            pl.BlockSpec((NC, _RB, DP), lambda i: (0, i, 0)),
            pl.BlockSpec((_RB, DP), lambda i: (i, 0)),
            pl.BlockSpec((DP, DP), lambda i: (0, 0)),
            pl.BlockSpec((DP, DP), lambda i: (0, 0)),
            pl.BlockSpec((1, DP), lambda i: (0, 0)),
        ],
        out_specs=pl.BlockSpec((_RB, DP), lambda i: (i, 0)),
        out_shape=jax.ShapeDtypeStruct((N, DP), jnp.float32),
    )(p, y, wr, wo, b)


def _tc_head_body(q_ref, h_ref, wr_ref, wo_ref, b_ref, o_ref):
    agg = q_ref[0] + q_ref[1]
    h2 = (jnp.dot(agg, wr_ref[...], preferred_element_type=jnp.float32)
          + jnp.dot(h_ref[...], wo_ref[...], preferred_element_type=jnp.float32)
          + b_ref[...])
    h6 = h2[:, :D]
    m = jnp.max(h6, axis=1, keepdims=True)
    ids = lax.broadcasted_iota(jnp.int32, h6.shape, 1)
    cand = jnp.where(h6 >= m, ids, D)
    sel = jnp.min(cand, axis=1, keepdims=True)
    o_ref[...] = jnp.where(ids == sel, 1.0, 0.0).astype(jnp.float32)


def _tc_head(q, h, wr, wo, b):
    return pl.pallas_call(
        _tc_head_body,
        grid=(N // _RB,),
        in_specs=[
            pl.BlockSpec((NC, _RB, DP), lambda i: (0, i, 0)),
            pl.BlockSpec((_RB, DP), lambda i: (i, 0)),
            pl.BlockSpec((DP, DP), lambda i: (0, 0)),
            pl.BlockSpec((DP, DP), lambda i: (0, 0)),
            pl.BlockSpec((1, DP), lambda i: (0, 0)),
        ],
        out_specs=pl.BlockSpec((_RB, D), lambda i: (i, 0)),
        out_shape=jax.ShapeDtypeStruct((N, D), jnp.float32),
    )(q, h, wr, wo, b)


def _pad_w(w):
    return jnp.pad(w.astype(jnp.float32), ((0, DP - D), (0, DP - D)))


def kernel(x, edge_index, W_root_0, W_rel_0, b_rel_0,
           W_root_1, W_rel_1, b_rel_1):
    ei = edge_index.astype(jnp.int32).reshape(2, NSUB, SUB)
    xe = jnp.pad(x.astype(jnp.float32), ((0, 0), (0, DP - D)))
    z = jnp.zeros((XPT, DP), jnp.float32)
    wr0, wo0 = _pad_w(W_rel_0), _pad_w(W_root_0)
    wr1, wo1 = _pad_w(W_rel_1), _pad_w(W_root_1)
    b0 = jnp.pad(b_rel_0.astype(jnp.float32), (0, DP - D)).reshape(1, DP)
    b1 = jnp.pad(b_rel_1.astype(jnp.float32), (0, DP - D)).reshape(1, DP)

    p = _sc_segsum(xe, ei, z)
    h1 = _tc_linear(p, xe, wr0, wo0, b0)
    q = _sc_segsum(h1, ei, z)
    return _tc_head(q, h1, wr1, wo1, b1)


# speedup vs baseline: 47.0991x; 1.0787x over previous
import functools

import jax
import jax.numpy as jnp
from jax import lax
from jax.experimental import pallas as pl
from jax.experimental.pallas import tpu as pltpu
from jax.experimental.pallas import tpu_sc as plsc

N = 100000
E = 3200000
D = 6
DP = 8
NC = 2
NS = 16
NW = NC * NS
SUB = 128
K = 8
NSUB = E // SUB
CPW = NSUB // NW
REM = NSUB % NW
F = CPW // K
XPT = N // NS


def _sc_body(x_hbm, ei_hbm, z_hbm, out_hbm, xs, acc, idx_s, idx_d, rows, gsem,
             ssem):
    cid = lax.axis_index("c")
    sid = lax.axis_index("s")
    wid = sid * NC + cid
    pltpu.sync_copy(x_hbm.at[pl.ds(sid * XPT, XPT)],
                    xs.at[pl.ds(sid * XPT, XPT)])
    pltpu.sync_copy(z_hbm, acc.at[pl.ds(sid * XPT, XPT)])
    plsc.subcore_barrier()

    cnt = CPW + jnp.where(wid < REM, 1, 0)
    off = wid * CPW + jnp.minimum(wid, REM)

    def outer(g, carry):
        r0 = off + g * K
        pltpu.sync_copy(ei_hbm.at[0, pl.ds(r0, K)], idx_s)
        pltpu.sync_copy(ei_hbm.at[1, pl.ds(r0, K)], idx_d)
        cps = [pltpu.async_copy(xs.at[idx_s.at[j]], rows.at[j], gsem)
               for j in range(K)]
        for cp in cps:
            cp.wait()
        scs = [pltpu.async_copy(rows.at[j], acc.at[idx_d.at[j]], ssem,
                                add=True)
               for j in range(K)]
        for sc in scs:
            sc.wait()
        return carry

    lax.fori_loop(0, F, outer, 0)

    r0t = off + cnt - K
    th = (F + 1) * K - cnt
    pltpu.sync_copy(ei_hbm.at[0, pl.ds(r0t, K)], idx_s)
    pltpu.sync_copy(ei_hbm.at[1, pl.ds(r0t, K)], idx_d)
    for j in range(K):
        @pl.when(j >= th)
        def _():
            pltpu.async_copy(xs.at[idx_s.at[j]], rows.at[j], gsem).wait()
            pltpu.sync_copy(rows.at[j], acc.at[idx_d.at[j]], add=True)

    plsc.subcore_barrier()
    pltpu.sync_copy(acc.at[pl.ds(sid * XPT, XPT)],
                    out_hbm.at[cid, pl.ds(sid * XPT, XPT)])


_sc_segsum = functools.partial(
    pl.kernel,
    out_type=jax.ShapeDtypeStruct((NC, N, DP), jnp.float32),
    mesh=plsc.VectorSubcoreMesh(core_axis_name="c", subcore_axis_name="s",
                                num_cores=NC, num_subcores=NS),
    scratch_types=[
        pltpu.VMEM_SHARED((N, DP), jnp.float32),
        pltpu.VMEM_SHARED((N, DP), jnp.float32),
        pltpu.VMEM((K, SUB), jnp.int32),
        pltpu.VMEM((K, SUB), jnp.int32),
        pltpu.VMEM((K, SUB, DP), jnp.float32),
        pltpu.SemaphoreType.DMA,
        pltpu.SemaphoreType.DMA,
    ],
    compiler_params=pltpu.CompilerParams(use_tc_tiling_on_sc=False),
)(_sc_body)


_RB = 10000


def _tc_linear_body(p_ref, y_ref, wr_ref, wo_ref, b_ref, o_ref):
    agg = p_ref[0] + p_ref[1]
    o_ref[...] = (
        jnp.dot(agg, wr_ref[...], preferred_element_type=jnp.float32)
        + jnp.dot(y_ref[...], wo_ref[...], preferred_element_type=jnp.float32)
        + b_ref[...]
    )


def _tc_linear(p, y, wr, wo, b):
    return pl.pallas_call(
        _tc_linear_body,
        grid=(N // _RB,),
        in_specs=[
            pl.BlockSpec((NC, _RB, DP), lambda i: (0, i, 0)),
            pl.BlockSpec((_RB, DP), lambda i: (i, 0)),
            pl.BlockSpec((DP, DP), lambda i: (0, 0)),
            pl.BlockSpec((DP, DP), lambda i: (0, 0)),
            pl.BlockSpec((1, DP), lambda i: (0, 0)),
        ],
        out_specs=pl.BlockSpec((_RB, DP), lambda i: (i, 0)),
        out_shape=jax.ShapeDtypeStruct((N, DP), jnp.float32),
    )(p, y, wr, wo, b)


def _tc_head_body(q_ref, h_ref, wr_ref, wo_ref, b_ref, o_ref):
    agg = q_ref[0] + q_ref[1]
    h2 = (jnp.dot(agg, wr_ref[...], preferred_element_type=jnp.float32)
          + jnp.dot(h_ref[...], wo_ref[...], preferred_element_type=jnp.float32)
          + b_ref[...])
    h6 = h2[:, :D]
    m = jnp.max(h6, axis=1, keepdims=True)
    ids = lax.broadcasted_iota(jnp.int32, h6.shape, 1)
    cand = jnp.where(h6 >= m, ids, D)
    sel = jnp.min(cand, axis=1, keepdims=True)
    o_ref[...] = jnp.where(ids == sel, 1.0, 0.0).astype(jnp.float32)


def _tc_head(q, h, wr, wo, b):
    return pl.pallas_call(
        _tc_head_body,
        grid=(N // _RB,),
        in_specs=[
            pl.BlockSpec((NC, _RB, DP), lambda i: (0, i, 0)),
            pl.BlockSpec((_RB, DP), lambda i: (i, 0)),
            pl.BlockSpec((DP, DP), lambda i: (0, 0)),
            pl.BlockSpec((DP, DP), lambda i: (0, 0)),
            pl.BlockSpec((1, DP), lambda i: (0, 0)),
        ],
        out_specs=pl.BlockSpec((_RB, D), lambda i: (i, 0)),
        out_shape=jax.ShapeDtypeStruct((N, D), jnp.float32),
    )(q, h, wr, wo, b)


def _pad_w(w):
    return jnp.pad(w.astype(jnp.float32), ((0, DP - D), (0, DP - D)))


def kernel(x, edge_index, W_root_0, W_rel_0, b_rel_0,
           W_root_1, W_rel_1, b_rel_1):
    ei = edge_index.astype(jnp.int32).reshape(2, NSUB, SUB)
    xe = jnp.pad(x.astype(jnp.float32), ((0, 0), (0, DP - D)))
    z = jnp.zeros((XPT, DP), jnp.float32)
    wr0, wo0 = _pad_w(W_rel_0), _pad_w(W_root_0)
    wr1, wo1 = _pad_w(W_rel_1), _pad_w(W_root_1)
    b0 = jnp.pad(b_rel_0.astype(jnp.float32), (0, DP - D)).reshape(1, DP)
    b1 = jnp.pad(b_rel_1.astype(jnp.float32), (0, DP - D)).reshape(1, DP)

    p = _sc_segsum(xe, ei, z)
    h1 = _tc_linear(p, xe, wr0, wo0, b0)
    q = _sc_segsum(h1, ei, z)
    return _tc_head(q, h1, wr1, wo1, b1)


# speedup vs baseline: 52.0020x; 1.1041x over previous
import functools

import jax
import jax.numpy as jnp
from jax import lax
from jax.experimental import pallas as pl
from jax.experimental.pallas import tpu as pltpu
from jax.experimental.pallas import tpu_sc as plsc

N = 100000
E = 3200000
D = 6
DP = 8
NC = 2
NS = 16
NW = NC * NS
SUB = 128
K = 8
NSUB = E // SUB
CPW = NSUB // NW
REM = NSUB % NW
F = CPW // K
XPT = N // NS


def _sc_body(x_hbm, ei_hbm, z_hbm, dz_hbm, out_hbm, xs, acc, idx_s, idx_d,
             rows, gsem, ssem0, ssem1):
    cid = lax.axis_index("c")
    sid = lax.axis_index("s")
    wid = sid * NC + cid
    pltpu.sync_copy(x_hbm.at[pl.ds(sid * XPT, XPT)],
                    xs.at[pl.ds(sid * XPT, XPT)])
    pltpu.sync_copy(z_hbm, acc.at[pl.ds(sid * XPT, XPT)])
    plsc.subcore_barrier()

    cnt = CPW + jnp.where(wid < REM, 1, 0)
    off = wid * CPW + jnp.minimum(wid, REM)
    ssems = (ssem0, ssem1)

    def sub_iter(g, b, first):
        r0 = off + g * K
        pltpu.sync_copy(ei_hbm.at[0, pl.ds(r0, K)], idx_s.at[b])
        pltpu.sync_copy(ei_hbm.at[1, pl.ds(r0, K)], idx_d.at[b])
        cps = [pltpu.async_copy(xs.at[idx_s.at[b, j]], rows.at[b, j], gsem)
               for j in range(K)]
        for cp in cps:
            cp.wait()
        if not first:
            pltpu.make_async_copy(dz_hbm, rows.at[1 - b],
                                  ssems[1 - b]).wait()
        for j in range(K):
            pltpu.async_copy(rows.at[b, j], acc.at[idx_d.at[b, j]],
                             ssems[b], add=True)

    sub_iter(0, 0, True)

    def pair(p, carry):
        sub_iter(2 * p + 1, 1, False)
        sub_iter(2 * p + 2, 0, False)
        return carry

    lax.fori_loop(0, (F - 1) // 2, pair, 0)

    r0t = off + cnt - K
    th = (F + 1) * K - cnt
    pltpu.sync_copy(ei_hbm.at[0, pl.ds(r0t, K)], idx_s.at[1])
    pltpu.sync_copy(ei_hbm.at[1, pl.ds(r0t, K)], idx_d.at[1])
    for j in range(K):
        @pl.when(j >= th)
        def _():
            pltpu.async_copy(xs.at[idx_s.at[1, j]], rows.at[1, j],
                             gsem).wait()
            pltpu.sync_copy(rows.at[1, j], acc.at[idx_d.at[1, j]], add=True)
    pltpu.make_async_copy(dz_hbm, rows.at[0], ssem0).wait()

    plsc.subcore_barrier()
    pltpu.sync_copy(acc.at[pl.ds(sid * XPT, XPT)],
                    out_hbm.at[cid, pl.ds(sid * XPT, XPT)])


_sc_segsum = functools.partial(
    pl.kernel,
    out_type=jax.ShapeDtypeStruct((NC, N, DP), jnp.float32),
    mesh=plsc.VectorSubcoreMesh(core_axis_name="c", subcore_axis_name="s",
                                num_cores=NC, num_subcores=NS),
    scratch_types=[
        pltpu.VMEM_SHARED((N, DP), jnp.float32),
        pltpu.VMEM_SHARED((N, DP), jnp.float32),
        pltpu.VMEM((2, K, SUB), jnp.int32),
        pltpu.VMEM((2, K, SUB), jnp.int32),
        pltpu.VMEM((2, K, SUB, DP), jnp.float32),
        pltpu.SemaphoreType.DMA,
        pltpu.SemaphoreType.DMA,
        pltpu.SemaphoreType.DMA,
    ],
    compiler_params=pltpu.CompilerParams(use_tc_tiling_on_sc=False),
)(_sc_body)


_RB = 10000


def _tc_linear_body(p_ref, y_ref, wr_ref, wo_ref, b_ref, o_ref):
    agg = p_ref[0] + p_ref[1]
    o_ref[...] = (
        jnp.dot(agg, wr_ref[...], preferred_element_type=jnp.float32)
        + jnp.dot(y_ref[...], wo_ref[...], preferred_element_type=jnp.float32)
        + b_ref[...]
    )


def _tc_linear(p, y, wr, wo, b):
    return pl.pallas_call(
        _tc_linear_body,
        grid=(N // _RB,),
        in_specs=[
            pl.BlockSpec((NC, _RB, DP), lambda i: (0, i, 0)),
            pl.BlockSpec((_RB, DP), lambda i: (i, 0)),
            pl.BlockSpec((DP, DP), lambda i: (0, 0)),
            pl.BlockSpec((DP, DP), lambda i: (0, 0)),
            pl.BlockSpec((1, DP), lambda i: (0, 0)),
        ],
        out_specs=pl.BlockSpec((_RB, DP), lambda i: (i, 0)),
        out_shape=jax.ShapeDtypeStruct((N, DP), jnp.float32),
    )(p, y, wr, wo, b)


def _tc_head_body(q_ref, h_ref, wr_ref, wo_ref, b_ref, o_ref):
    agg = q_ref[0] + q_ref[1]
    h2 = (jnp.dot(agg, wr_ref[...], preferred_element_type=jnp.float32)
          + jnp.dot(h_ref[...], wo_ref[...], preferred_element_type=jnp.float32)
          + b_ref[...])
    h6 = h2[:, :D]
    m = jnp.max(h6, axis=1, keepdims=True)
    ids = lax.broadcasted_iota(jnp.int32, h6.shape, 1)
    cand = jnp.where(h6 >= m, ids, D)
    sel = jnp.min(cand, axis=1, keepdims=True)
    o_ref[...] = jnp.where(ids == sel, 1.0, 0.0).astype(jnp.float32)


def _tc_head(q, h, wr, wo, b):
    return pl.pallas_call(
        _tc_head_body,
        grid=(N // _RB,),
        in_specs=[
            pl.BlockSpec((NC, _RB, DP), lambda i: (0, i, 0)),
            pl.BlockSpec((_RB, DP), lambda i: (i, 0)),
            pl.BlockSpec((DP, DP), lambda i: (0, 0)),
            pl.BlockSpec((DP, DP), lambda i: (0, 0)),
            pl.BlockSpec((1, DP), lambda i: (0, 0)),
        ],
        out_specs=pl.BlockSpec((_RB, D), lambda i: (i, 0)),
        out_shape=jax.ShapeDtypeStruct((N, D), jnp.float32),
    )(q, h, wr, wo, b)


def _pad_w(w):
    return jnp.pad(w.astype(jnp.float32), ((0, DP - D), (0, DP - D)))


def kernel(x, edge_index, W_root_0, W_rel_0, b_rel_0,
           W_root_1, W_rel_1, b_rel_1):
    ei = edge_index.astype(jnp.int32).reshape(2, NSUB, SUB)
    xe = jnp.pad(x.astype(jnp.float32), ((0, 0), (0, DP - D)))
    z = jnp.zeros((XPT, DP), jnp.float32)
    dz = jnp.zeros((K, SUB, DP), jnp.float32)
    wr0, wo0 = _pad_w(W_rel_0), _pad_w(W_root_0)
    wr1, wo1 = _pad_w(W_rel_1), _pad_w(W_root_1)
    b0 = jnp.pad(b_rel_0.astype(jnp.float32), (0, DP - D)).reshape(1, DP)
    b1 = jnp.pad(b_rel_1.astype(jnp.float32), (0, DP - D)).reshape(1, DP)

    p = _sc_segsum(xe, ei, z, dz)
    h1 = _tc_linear(p, xe, wr0, wo0, b0)
    q = _sc_segsum(h1, ei, z, dz)
    return _tc_head(q, h1, wr1, wo1, b1)


# speedup vs baseline: 71.6159x; 1.3772x over previous
import functools

import jax
import jax.numpy as jnp
from jax import lax
from jax.experimental import pallas as pl
from jax.experimental.pallas import tpu as pltpu
from jax.experimental.pallas import tpu_sc as plsc

N = 100000
E = 3200000
D = 6
DP = 8
NC = 2
NS = 16
NW = NC * NS
SUB = 128
K = 8
NSUB = E // SUB
CPW = NSUB // NW
REM = NSUB % NW
F = CPW // K
XPT = N // NS


def _sc_body(x_hbm, ei_hbm, z_hbm, dz_hbm, out_hbm, xs, acc, idx_s, idx_d,
             rows, gsem, ssem0, ssem1):
    cid = lax.axis_index("c")
    sid = lax.axis_index("s")
    wid = sid * NC + cid
    pltpu.sync_copy(x_hbm.at[pl.ds(sid * XPT, XPT)],
                    xs.at[pl.ds(sid * XPT, XPT)])
    pltpu.sync_copy(z_hbm, acc.at[pl.ds(sid * XPT, XPT)])
    plsc.subcore_barrier()

    cnt = CPW + jnp.where(wid < REM, 1, 0)
    off = wid * CPW + jnp.minimum(wid, REM)
    ssems = (ssem0, ssem1)

    def sub_iter(g, b, first):
        r0 = off + g * K
        pltpu.sync_copy(ei_hbm.at[0, pl.ds(r0, K)], idx_s.at[b])
        pltpu.sync_copy(ei_hbm.at[1, pl.ds(r0, K)], idx_d.at[b])
        cps = [pltpu.async_copy(xs.at[idx_s.at[b, j]], rows.at[b, j], gsem)
               for j in range(K)]
        for cp in cps:
            cp.wait()
        if not first:
            pltpu.make_async_copy(dz_hbm, rows.at[1 - b],
                                  ssems[1 - b]).wait()
        for j in range(K):
            pltpu.async_copy(rows.at[b, j], acc.at[idx_d.at[b, j]],
                             ssems[b], add=True)

    sub_iter(0, 0, True)

    def pair(p, carry):
        sub_iter(2 * p + 1, 1, False)
        sub_iter(2 * p + 2, 0, False)
        return carry

    lax.fori_loop(0, (F - 1) // 2, pair, 0)

    r0t = off + cnt - K
    th = (F + 1) * K - cnt
    pltpu.sync_copy(ei_hbm.at[0, pl.ds(r0t, K)], idx_s.at[1])
    pltpu.sync_copy(ei_hbm.at[1, pl.ds(r0t, K)], idx_d.at[1])
    for j in range(K):
        @pl.when(j >= th)
        def _():
            pltpu.async_copy(xs.at[idx_s.at[1, j]], rows.at[1, j],
                             gsem).wait()
            pltpu.sync_copy(rows.at[1, j], acc.at[idx_d.at[1, j]], add=True)
    pltpu.make_async_copy(dz_hbm, rows.at[0], ssem0).wait()

    plsc.subcore_barrier()
    pltpu.sync_copy(acc.at[pl.ds(sid * XPT, XPT)],
                    out_hbm.at[cid, pl.ds(sid * XPT, XPT)])


_sc_segsum = functools.partial(
    pl.kernel,
    out_type=jax.ShapeDtypeStruct((NC, N, DP), jnp.float32),
    mesh=plsc.VectorSubcoreMesh(core_axis_name="c", subcore_axis_name="s",
                                num_cores=NC, num_subcores=NS),
    scratch_types=[
        pltpu.VMEM_SHARED((N, DP), jnp.float32),
        pltpu.VMEM_SHARED((N, DP), jnp.float32),
        pltpu.VMEM((2, K, SUB), jnp.int32),
        pltpu.VMEM((2, K, SUB), jnp.int32),
        pltpu.VMEM((2, K, SUB, DP), jnp.float32),
        pltpu.SemaphoreType.DMA,
        pltpu.SemaphoreType.DMA,
        pltpu.SemaphoreType.DMA,
    ],
    compiler_params=pltpu.CompilerParams(use_tc_tiling_on_sc=False),
)(_sc_body)


NP = N * DP // 128


def _tc_linear_body(p_ref, y_ref, kwr_ref, kwo_ref, b_ref, o_ref):
    agg = p_ref[0] + p_ref[1]
    o_ref[...] = (
        jnp.dot(agg, kwr_ref[...], preferred_element_type=jnp.float32)
        + jnp.dot(y_ref[...], kwo_ref[...], preferred_element_type=jnp.float32)
        + b_ref[...]
    )


def _tc_linear(p, y, kwr, kwo, b):
    return pl.pallas_call(
        _tc_linear_body,
        out_shape=jax.ShapeDtypeStruct((NP, 128), jnp.float32),
    )(p, y, kwr, kwo, b)


def _tc_head_body(q_ref, h_ref, kwr_ref, kwo_ref, b_ref, o_ref):
    agg = q_ref[0] + q_ref[1]
    h2 = (jnp.dot(agg, kwr_ref[...], preferred_element_type=jnp.float32)
          + jnp.dot(h_ref[...], kwo_ref[...], preferred_element_type=jnp.float32)
          + b_ref[...])
    jl = lax.broadcasted_iota(jnp.int32, h2.shape, 1) % DP
    ok = jl < D
    for d in [dd for dd in range(-5, 6) if dd != 0]:
        nb = pltpu.roll(h2, (-d) % 128, 1)
        valid = (jl + d >= 0) & (jl + d < D)
        cmp = (h2 > nb) if d < 0 else (h2 >= nb)
        ok = ok & (cmp | jnp.logical_not(valid))
    o_ref[...] = jnp.where(ok, 1.0, 0.0).astype(jnp.float32)


def _tc_head(q, h, kwr, kwo, b):
    return pl.pallas_call(
        _tc_head_body,
        out_shape=jax.ShapeDtypeStruct((NP, 128), jnp.float32),
    )(q, h, kwr, kwo, b)


def _pad_w(w):
    return jnp.pad(w.astype(jnp.float32), ((0, DP - D), (0, DP - D)))


def _kron_w(w):
    return jnp.kron(jnp.eye(16, dtype=jnp.float32), _pad_w(w))


def _tile_b(b):
    return jnp.tile(jnp.pad(b.astype(jnp.float32), (0, DP - D)),
                    16).reshape(1, 128)


def kernel(x, edge_index, W_root_0, W_rel_0, b_rel_0,
           W_root_1, W_rel_1, b_rel_1):
    ei = edge_index.astype(jnp.int32).reshape(2, NSUB, SUB)
    xp = jnp.pad(x.astype(jnp.float32), ((0, 0), (0, DP - D))).reshape(NP, 128)
    z = jnp.zeros((XPT, DP), jnp.float32)
    dz = jnp.zeros((K, SUB, DP), jnp.float32)
    kwr0, kwo0 = _kron_w(W_rel_0), _kron_w(W_root_0)
    kwr1, kwo1 = _kron_w(W_rel_1), _kron_w(W_root_1)
    b0t, b1t = _tile_b(b_rel_0), _tile_b(b_rel_1)

    p = _sc_segsum(xp.reshape(N, DP), ei, z, dz)
    h1p = _tc_linear(p.reshape(NC, NP, 128), xp, kwr0, kwo0, b0t)
    q = _sc_segsum(h1p.reshape(N, DP), ei, z, dz)
    outp = _tc_head(q.reshape(NC, NP, 128), h1p, kwr1, kwo1, b1t)
    return outp.reshape(N, DP)[:, :D]
